# Initial kernel scaffold; baseline (speedup 1.0000x reference)
import jax
import jax.numpy as jnp
from jax.experimental import pallas as pl


def kernel(edge_type, pe):
    raise NotImplementedError("write your pallas kernel here")


# speedup vs baseline: 3.3275x; 3.3275x over previous
import functools

import jax
import jax.numpy as jnp
from jax import lax
from jax.experimental import pallas as pl
from jax.experimental.pallas import tpu as pltpu
from jax.experimental.pallas import tpu_sc as plsc

D_MODEL = 128
NUM_CORES = 2
NUM_SUBCORES = 16
NUM_WORKERS = NUM_CORES * NUM_SUBCORES
CHUNK = 128
NBUF = 2

_mesh = plsc.VectorSubcoreMesh(
    core_axis_name="c",
    subcore_axis_name="s",
    num_cores=NUM_CORES,
    num_subcores=NUM_SUBCORES,
)


def _make_kernel(total_idx):
  per_w = total_idx // NUM_WORKERS
  nch = per_w // CHUNK

  @functools.partial(
      pl.kernel,
      out_type=jax.ShapeDtypeStruct((total_idx, D_MODEL), jnp.float32),
      mesh=_mesh,
      scratch_types=[
          pltpu.VMEM((nch, CHUNK), jnp.int32),
          [pltpu.VMEM((CHUNK, D_MODEL), jnp.float32) for _ in range(NBUF)],
          [pltpu.SemaphoreType.DMA for _ in range(NBUF)],
          [pltpu.SemaphoreType.DMA for _ in range(NBUF)],
      ],
  )
  def gather_kernel(idx_hbm, table_hbm, out_hbm, idx_v, rows, gsem, osem):
    wid = lax.axis_index("s") * NUM_CORES + lax.axis_index("c")
    base = wid * per_w

    pltpu.sync_copy(idx_hbm.at[wid], idx_v)

    def gather_start(c, b):
      pltpu.async_copy(table_hbm.at[idx_v.at[c]], rows[b], gsem[b])

    def gather_wait(b):
      pltpu.make_async_copy(table_hbm.at[idx_v.at[0]], rows[b], gsem[b]).wait()

    def out_start(c, b):
      pltpu.async_copy(rows[b], out_hbm.at[pl.ds(base + c * CHUNK, CHUNK)],
                       osem[b])

    def out_wait(b):
      pltpu.make_async_copy(rows[b], out_hbm.at[pl.ds(base, CHUNK)],
                            osem[b]).wait()

    for b in range(NBUF):
      gather_start(b, b)

    @pl.loop(0, nch, step=NBUF)
    def _(j):
      for b in range(NBUF):
        c = j + b
        gather_wait(b)
        out_start(c, b)

        @pl.when(c + NBUF < nch)
        def _():
          out_wait(b)
          gather_start(c + NBUF, b)

    for b in range(NBUF):
      out_wait(b)

  return gather_kernel


@jax.jit
def kernel(edge_type, pe):
  n, k = edge_type.shape
  total = n * k
  idx = edge_type.reshape(NUM_WORKERS, total // (NUM_WORKERS * CHUNK), CHUNK)
  out = _make_kernel(total)(idx, pe)
  return out.reshape(n, k, D_MODEL)


# speedup vs baseline: 3.3480x; 1.0061x over previous
import functools

import jax
import jax.numpy as jnp
from jax import lax
from jax.experimental import pallas as pl
from jax.experimental.pallas import tpu as pltpu
from jax.experimental.pallas import tpu_sc as plsc

D_MODEL = 128
NUM_CORES = 2
NUM_SUBCORES = 16
NUM_WORKERS = NUM_CORES * NUM_SUBCORES
CHUNK = 128
NBUF = 5

_mesh = plsc.VectorSubcoreMesh(
    core_axis_name="c",
    subcore_axis_name="s",
    num_cores=NUM_CORES,
    num_subcores=NUM_SUBCORES,
)


def _make_kernel(total_idx):
  per_w = total_idx // NUM_WORKERS
  nch = per_w // CHUNK

  @functools.partial(
      pl.kernel,
      out_type=jax.ShapeDtypeStruct((total_idx, D_MODEL), jnp.float32),
      mesh=_mesh,
      scratch_types=[
          pltpu.VMEM((nch, CHUNK), jnp.int32),
          [pltpu.VMEM((CHUNK, D_MODEL), jnp.float32) for _ in range(NBUF)],
          [pltpu.SemaphoreType.DMA for _ in range(NBUF)],
          [pltpu.SemaphoreType.DMA for _ in range(NBUF)],
      ],
  )
  def gather_kernel(idx_hbm, table_hbm, out_hbm, idx_v, rows, gsem, osem):
    wid = lax.axis_index("s") * NUM_CORES + lax.axis_index("c")
    base = wid * per_w

    pltpu.sync_copy(idx_hbm.at[wid], idx_v)

    def gather_start(c, b):
      pltpu.async_copy(table_hbm.at[idx_v.at[c]], rows[b], gsem[b])

    def gather_wait(b):
      pltpu.make_async_copy(table_hbm.at[idx_v.at[0]], rows[b], gsem[b]).wait()

    def out_start(c, b):
      pltpu.async_copy(rows[b], out_hbm.at[pl.ds(base + c * CHUNK, CHUNK)],
                       osem[b])

    def out_wait(b):
      pltpu.make_async_copy(rows[b], out_hbm.at[pl.ds(base, CHUNK)],
                            osem[b]).wait()

    for b in range(NBUF):
      gather_start(b, b)

    @pl.loop(0, nch, step=NBUF)
    def _(j):
      for b in range(NBUF):
        c = j + b
        gather_wait(b)
        out_start(c, b)

        @pl.when(c + NBUF < nch)
        def _():
          out_wait(b)
          gather_start(c + NBUF, b)

    for b in range(NBUF):
      out_wait(b)

  return gather_kernel


@jax.jit
def kernel(edge_type, pe):
  n, k = edge_type.shape
  total = n * k
  idx = edge_type.reshape(NUM_WORKERS, total // (NUM_WORKERS * CHUNK), CHUNK)
  out = _make_kernel(total)(idx, pe)
  return out.reshape(n, k, D_MODEL)


# speedup vs baseline: 5.9318x; 1.7718x over previous
import functools

import jax
import jax.numpy as jnp
from jax import lax
from jax.experimental import pallas as pl
from jax.experimental.pallas import tpu as pltpu
from jax.experimental.pallas import tpu_sc as plsc

D_MODEL = 128
NUM_CORES = 2
NUM_SUBCORES = 16
NUM_WORKERS = NUM_CORES * NUM_SUBCORES
NI = 4
NBUF = 2

_mesh = plsc.VectorSubcoreMesh(
    core_axis_name="c",
    subcore_axis_name="s",
    num_cores=NUM_CORES,
    num_subcores=NUM_SUBCORES,
)


def _make_kernel(n, k):
  per_w = n // NUM_WORKERS
  nblk = per_w // NI

  @functools.partial(
      pl.kernel,
      out_type=jax.ShapeDtypeStruct((n, k, D_MODEL), jnp.float32),
      mesh=_mesh,
      compiler_params=pltpu.CompilerParams(use_tc_tiling_on_sc=True),
      scratch_types=[
          pltpu.VMEM((per_w, k), jnp.int32),
          [pltpu.VMEM((NI, k, D_MODEL), jnp.float32) for _ in range(NBUF)],
          [pltpu.SemaphoreType.DMA for _ in range(NBUF)],
          [pltpu.SemaphoreType.DMA for _ in range(NBUF)],
      ],
  )
  def gather_kernel(idx_hbm, table_hbm, out_hbm, idx_v, rows, gsem, osem):
    wid = lax.axis_index("s") * NUM_CORES + lax.axis_index("c")
    i0 = wid * per_w

    pltpu.sync_copy(idx_hbm.at[pl.ds(i0, per_w)], idx_v)

    def gathers_start(g, b):
      for r in range(NI):
        pltpu.async_copy(table_hbm.at[idx_v.at[g * NI + r]], rows[b].at[r],
                         gsem[b])

    def gathers_wait(b):
      for r in range(NI):
        pltpu.make_async_copy(table_hbm.at[idx_v.at[0]], rows[b].at[r],
                              gsem[b]).wait()

    def out_start(g, b):
      pltpu.async_copy(rows[b], out_hbm.at[pl.ds(i0 + g * NI, NI)], osem[b])

    def out_wait(b):
      pltpu.make_async_copy(rows[b], out_hbm.at[pl.ds(i0, NI)], osem[b]).wait()

    for b in range(NBUF):
      gathers_start(b, b)

    @pl.loop(0, nblk, step=NBUF)
    def _(g):
      for b in range(NBUF):
        blk = g + b
        gathers_wait(b)
        out_start(blk, b)

        @pl.when(blk + NBUF < nblk)
        def _():
          out_wait(b)
          gathers_start(blk + NBUF, b)

    for b in range(NBUF):
      out_wait(b)

  return gather_kernel


@jax.jit
def kernel(edge_type, pe):
  n, k = edge_type.shape
  return _make_kernel(n, k)(edge_type, pe)
